# baseline (speedup 1.0000x reference)
import functools

import jax
import jax.numpy as jnp
from jax import lax
from jax.experimental import pallas as pl
from jax.experimental.pallas import tpu as pltpu
from jax.experimental.pallas import tpu_sc as plsc

N = 10000
E = 320000
D = 128
C = 10
G = 64

NPAD = 10240
NC, NS = 2, 16
NW = NC * NS
CHUNK = 128
CPW = 80
NCHUNKS_P = NW * CPW
EPAD = NCHUNKS_P * CHUNK
ROWS_PER_SUB = NPAD // NS
BLK = 1000
GRID = N // BLK

_mesh = plsc.VectorSubcoreMesh(core_axis_name="c", subcore_axis_name="s")


def _worker_id():
    return lax.axis_index("s") * NC + lax.axis_index("c")


@functools.partial(
    pl.kernel,
    out_type=jax.ShapeDtypeStruct((NC * NPAD,), jnp.float32),
    mesh=_mesh,
    scratch_types=[
        pltpu.VMEM((CPW, CHUNK), jnp.int32),
        pltpu.VMEM((CHUNK,), jnp.float32),
        pltpu.VMEM((ROWS_PER_SUB,), jnp.float32),
        pltpu.VMEM_SHARED((NPAD,), jnp.float32),
    ],
)
def _sc_degree(dst_hbm, out_hbm, didx_v, ones_v, zbuf_v, cnt_sh):
    cid = lax.axis_index("c")
    sid = lax.axis_index("s")
    w = _worker_id()

    def fill(i, _):
        zbuf_v[pl.ds(i * 16, 16)] = jnp.zeros((16,), jnp.float32)
        ones_v[pl.ds((i % 8) * 16, 16)] = jnp.full((16,), 1.0, jnp.float32)
        return 0

    lax.fori_loop(0, ROWS_PER_SUB // 16, fill, 0)
    pltpu.sync_copy(zbuf_v, cnt_sh.at[pl.ds(sid * ROWS_PER_SUB, ROWS_PER_SUB)])
    pltpu.sync_copy(dst_hbm.at[pl.ds(w * CPW, CPW)], didx_v)
    plsc.subcore_barrier()

    def body(c, _):
        pltpu.sync_copy(ones_v, cnt_sh.at[didx_v.at[c]], add=True)
        return 0

    lax.fori_loop(0, CPW, body, 0)
    plsc.subcore_barrier()
    pltpu.sync_copy(
        cnt_sh.at[pl.ds(sid * ROWS_PER_SUB, ROWS_PER_SUB)],
        out_hbm.at[pl.ds(cid * NPAD + sid * ROWS_PER_SUB, ROWS_PER_SUB)],
    )


@functools.partial(
    pl.kernel,
    out_type=jax.ShapeDtypeStruct((NC * NPAD, D), jnp.float32),
    mesh=_mesh,
    scratch_types=[
        pltpu.VMEM((CHUNK,), jnp.int32),
        pltpu.VMEM((CHUNK,), jnp.int32),
        pltpu.VMEM((CHUNK, D), jnp.float32),
        pltpu.VMEM((CHUNK, D), jnp.float32),
        pltpu.VMEM_SHARED((NPAD, D), jnp.float32),
        pltpu.SemaphoreType.DMA,
    ],
)
def _sc_msgpass(y_hbm, src_hbm, dst_hbm, out_hbm,
                sidx_v, didx_v, rows_v, zbuf_v, acc_sh, sem):
    cid = lax.axis_index("c")
    sid = lax.axis_index("s")
    w = _worker_id()

    def fill(r, _):
        for c8 in range(D // 16):
            zbuf_v[r, pl.ds(c8 * 16, 16)] = jnp.zeros((16,), jnp.float32)
        return 0

    lax.fori_loop(0, CHUNK, fill, 0)
    for k in range(ROWS_PER_SUB // CHUNK):
        pltpu.sync_copy(
            zbuf_v, acc_sh.at[pl.ds(sid * ROWS_PER_SUB + k * CHUNK, CHUNK)])
    plsc.subcore_barrier()

    def body(i, _):
        base = (w * CPW + i) * CHUNK
        pltpu.sync_copy(src_hbm.at[pl.ds(base, CHUNK)], sidx_v)
        pltpu.sync_copy(dst_hbm.at[pl.ds(base, CHUNK)], didx_v)
        pltpu.async_copy(y_hbm.at[sidx_v], rows_v, sem).wait()
        pltpu.sync_copy(rows_v, acc_sh.at[didx_v], add=True)
        return 0

    lax.fori_loop(0, CPW, body, 0)
    plsc.subcore_barrier()
    for k in range(ROWS_PER_SUB // CHUNK):
        pltpu.sync_copy(
            acc_sh.at[pl.ds(sid * ROWS_PER_SUB + k * CHUNK, CHUNK)],
            out_hbm.at[pl.ds(cid * NPAD + sid * ROWS_PER_SUB + k * CHUNK,
                             CHUNK)],
        )


def _tc1_body(c0_ref, c1_ref, x_ref, w1_ref, y1_ref, dinv_ref):
    deg = c0_ref[...] + c1_ref[...] + 1.0
    dinv = lax.rsqrt(deg)
    xw = jnp.dot(x_ref[...], w1_ref[...], preferred_element_type=jnp.float32)
    y1_ref[...] = xw * dinv
    dinv_ref[...] = dinv


def _tc2_body(s1a_ref, s1b_ref, y1_ref, dinv_ref, b1_ref, w2_ref, y2_ref):
    dinv = dinv_ref[...]
    agg = s1a_ref[...] + s1b_ref[...] + y1_ref[...]
    h = jnp.maximum(agg * dinv + b1_ref[...], 0.0)
    y2_ref[...] = jnp.dot(h, w2_ref[...],
                          preferred_element_type=jnp.float32) * dinv


def _tc3a_body(s2a_ref, s2b_ref, y2_ref, dinv_ref, b2_ref, h2_ref, st_ref):
    i = pl.program_id(0)
    agg = s2a_ref[...] + s2b_ref[...] + y2_ref[...]
    h2 = jnp.maximum(agg * dinv_ref[...] + b2_ref[...], 0.0)
    h2_ref[...] = h2
    st = jnp.concatenate(
        [jnp.sum(h2, 0, keepdims=True), jnp.sum(h2 * h2, 0, keepdims=True)], 0)

    @pl.when(i == 0)
    def _():
        st_ref[...] = st

    @pl.when(i > 0)
    def _():
        st_ref[...] += st


def _tc3b_body(h2_ref, bat_ref, st_ref, g2_ref, beta2_ref, wfc_ref, bfc_ref,
               out_ref, pooled_s, cnt_s):
    i = pl.program_id(0)
    m = st_ref[0:1, :] * (1.0 / N)
    v = st_ref[1:2, :] * (1.0 / N) - m * m
    h2n = jnp.maximum(
        (h2_ref[...] - m) * lax.rsqrt(v + 1e-5) * g2_ref[...] + beta2_ref[...],
        0.0)
    oh = (bat_ref[...] == lax.broadcasted_iota(jnp.int32, (BLK, G), 1)
          ).astype(jnp.float32)
    p = lax.dot_general(oh, h2n, (((0,), (0,)), ((), ())),
                        preferred_element_type=jnp.float32)
    c = lax.dot_general(oh, jnp.ones((BLK, D), jnp.float32),
                        (((0,), (0,)), ((), ())),
                        preferred_element_type=jnp.float32)

    @pl.when(i == 0)
    def _():
        pooled_s[...] = p
        cnt_s[...] = c

    @pl.when(i > 0)
    def _():
        pooled_s[...] += p
        cnt_s[...] += c

    @pl.when(i == pl.num_programs(0) - 1)
    def _():
        pm = pooled_s[...] / jnp.maximum(cnt_s[...], 1.0)
        logits = jnp.dot(pm, wfc_ref[...],
                         preferred_element_type=jnp.float32) + bfc_ref[...]
        mx = jnp.max(logits, axis=1, keepdims=True)
        lse = jnp.log(jnp.sum(jnp.exp(logits - mx), 1, keepdims=True)) + mx
        out_ref[...] = logits - lse


def _row_spec(shape):
    return pl.BlockSpec(shape, lambda i: (i, 0))


def _const_spec(shape):
    return pl.BlockSpec(shape, lambda i: (0, 0))


def kernel(x, edge_index, batch, W1, b1, W2, b2, g2, beta2, Wfc, bfc):
    pad = EPAD - E
    srcp = jnp.concatenate(
        [edge_index[0], jnp.zeros((pad,), jnp.int32)]).reshape(NCHUNKS_P, CHUNK)
    dstp = jnp.concatenate(
        [edge_index[1], jnp.full((pad,), N, jnp.int32)]).reshape(NCHUNKS_P, CHUNK)

    srcf = srcp.reshape(EPAD)
    dstf = dstp.reshape(EPAD)
    cnt = _sc_degree(dstp)
    c0 = cnt[:N].reshape(N, 1)
    c1 = cnt[NPAD:NPAD + N].reshape(N, 1)

    y1, dinv = pl.pallas_call(
        _tc1_body,
        grid=(GRID,),
        in_specs=[_row_spec((BLK, 1)), _row_spec((BLK, 1)),
                  _row_spec((BLK, D)), _const_spec((D, D))],
        out_specs=[_row_spec((BLK, D)), _row_spec((BLK, 1))],
        out_shape=[jax.ShapeDtypeStruct((N, D), jnp.float32),
                   jax.ShapeDtypeStruct((N, 1), jnp.float32)],
    )(c0, c1, x, W1)

    s1 = _sc_msgpass(y1, srcf, dstf)

    y2 = pl.pallas_call(
        _tc2_body,
        grid=(GRID,),
        in_specs=[_row_spec((BLK, D)), _row_spec((BLK, D)),
                  _row_spec((BLK, D)), _row_spec((BLK, 1)),
                  _const_spec((1, D)), _const_spec((D, D))],
        out_specs=_row_spec((BLK, D)),
        out_shape=jax.ShapeDtypeStruct((N, D), jnp.float32),
    )(s1[:N], s1[NPAD:NPAD + N], y1, dinv, b1.reshape(1, D), W2)

    s2 = _sc_msgpass(y2, srcf, dstf)

    h2, st = pl.pallas_call(
        _tc3a_body,
        grid=(GRID,),
        in_specs=[_row_spec((BLK, D)), _row_spec((BLK, D)),
                  _row_spec((BLK, D)), _row_spec((BLK, 1)),
                  _const_spec((1, D))],
        out_specs=[_row_spec((BLK, D)), _const_spec((2, D))],
        out_shape=[jax.ShapeDtypeStruct((N, D), jnp.float32),
                   jax.ShapeDtypeStruct((2, D), jnp.float32)],
    )(s2[:N], s2[NPAD:NPAD + N], y2, dinv, b2.reshape(1, D))

    out = pl.pallas_call(
        _tc3b_body,
        grid=(GRID,),
        in_specs=[_row_spec((BLK, D)), _row_spec((BLK, 1)),
                  _const_spec((2, D)), _const_spec((1, D)),
                  _const_spec((1, D)), _const_spec((D, C)),
                  _const_spec((1, C))],
        out_specs=_const_spec((G, C)),
        out_shape=jax.ShapeDtypeStruct((G, C), jnp.float32),
        scratch_shapes=[pltpu.VMEM((G, D), jnp.float32),
                        pltpu.VMEM((G, D), jnp.float32)],
    )(h2, batch.reshape(N, 1), st, g2.reshape(1, D), beta2.reshape(1, D),
      Wfc, bfc.reshape(1, C))

    return out


# speedup vs baseline: 1.2150x; 1.2150x over previous
import functools

import jax
import jax.numpy as jnp
from jax import lax
from jax.experimental import pallas as pl
from jax.experimental.pallas import tpu as pltpu
from jax.experimental.pallas import tpu_sc as plsc

N = 10000
E = 320000
D = 128
C = 10
G = 64

NPAD = 10240
NC, NS = 2, 16
NW = NC * NS
CHUNK = 128
CPW = 80
NCHUNKS_P = NW * CPW
EPAD = NCHUNKS_P * CHUNK
ROWS_PER_SUB = NPAD // NS
BLK = 1000
GRID = N // BLK

_mesh = plsc.VectorSubcoreMesh(core_axis_name="c", subcore_axis_name="s")


def _worker_id():
    return lax.axis_index("s") * NC + lax.axis_index("c")


@functools.partial(
    pl.kernel,
    out_type=jax.ShapeDtypeStruct((NC * NPAD,), jnp.float32),
    mesh=_mesh,
    scratch_types=[
        pltpu.VMEM((CPW, CHUNK), jnp.int32),
        pltpu.VMEM((CHUNK,), jnp.float32),
        pltpu.VMEM((ROWS_PER_SUB,), jnp.float32),
        pltpu.VMEM_SHARED((NPAD,), jnp.float32),
    ],
)
def _sc_degree(dst_hbm, out_hbm, didx_v, ones_v, zbuf_v, cnt_sh):
    cid = lax.axis_index("c")
    sid = lax.axis_index("s")
    w = _worker_id()

    def fill(i, _):
        zbuf_v[pl.ds(i * 16, 16)] = jnp.zeros((16,), jnp.float32)
        ones_v[pl.ds((i % 8) * 16, 16)] = jnp.full((16,), 1.0, jnp.float32)
        return 0

    lax.fori_loop(0, ROWS_PER_SUB // 16, fill, 0)
    pltpu.sync_copy(zbuf_v, cnt_sh.at[pl.ds(sid * ROWS_PER_SUB, ROWS_PER_SUB)])
    pltpu.sync_copy(dst_hbm.at[pl.ds(w * CPW, CPW)], didx_v)
    plsc.subcore_barrier()

    def body(c, _):
        pltpu.sync_copy(ones_v, cnt_sh.at[didx_v.at[c]], add=True)
        return 0

    lax.fori_loop(0, CPW, body, 0)
    plsc.subcore_barrier()
    pltpu.sync_copy(
        cnt_sh.at[pl.ds(sid * ROWS_PER_SUB, ROWS_PER_SUB)],
        out_hbm.at[pl.ds(cid * NPAD + sid * ROWS_PER_SUB, ROWS_PER_SUB)],
    )


@functools.partial(
    pl.kernel,
    out_type=jax.ShapeDtypeStruct((NC * NPAD, D), jnp.float32),
    mesh=_mesh,
    scratch_types=[
        pltpu.VMEM((CHUNK,), jnp.int32),
        pltpu.VMEM((CHUNK,), jnp.int32),
        pltpu.VMEM((CHUNK, D), jnp.float32),
        pltpu.VMEM((CHUNK, D), jnp.float32),
        pltpu.VMEM_SHARED((NPAD, D), jnp.float32),
        pltpu.SemaphoreType.DMA,
    ],
)
def _sc_msgpass(y_hbm, src_hbm, dst_hbm, out_hbm,
                sidx_v, didx_v, rows_v, zbuf_v, acc_sh, sem):
    cid = lax.axis_index("c")
    sid = lax.axis_index("s")
    w = _worker_id()

    def fill(r, _):
        for c8 in range(D // 16):
            zbuf_v[r, pl.ds(c8 * 16, 16)] = jnp.zeros((16,), jnp.float32)
        return 0

    lax.fori_loop(0, CHUNK, fill, 0)
    for k in range(ROWS_PER_SUB // CHUNK):
        pltpu.sync_copy(
            zbuf_v, acc_sh.at[pl.ds(sid * ROWS_PER_SUB + k * CHUNK, CHUNK)])
    plsc.subcore_barrier()

    def body(i, _):
        base = (w + i * NW) * CHUNK
        pltpu.sync_copy(src_hbm.at[pl.ds(base, CHUNK)], sidx_v)
        pltpu.sync_copy(dst_hbm.at[pl.ds(base, CHUNK)], didx_v)
        pltpu.async_copy(y_hbm.at[sidx_v], rows_v, sem).wait()
        pltpu.sync_copy(rows_v, acc_sh.at[didx_v], add=True)
        return 0

    lax.fori_loop(0, CPW, body, 0)
    plsc.subcore_barrier()
    for k in range(ROWS_PER_SUB // CHUNK):
        pltpu.sync_copy(
            acc_sh.at[pl.ds(sid * ROWS_PER_SUB + k * CHUNK, CHUNK)],
            out_hbm.at[pl.ds(cid * NPAD + sid * ROWS_PER_SUB + k * CHUNK,
                             CHUNK)],
        )


def _tc1_body(c0_ref, c1_ref, x_ref, w1_ref, y1_ref, dinv_ref):
    deg = c0_ref[...] + c1_ref[...] + 1.0
    dinv = lax.rsqrt(deg)
    xw = jnp.dot(x_ref[...], w1_ref[...], preferred_element_type=jnp.float32)
    y1_ref[...] = xw * dinv
    dinv_ref[...] = dinv


def _tc2_body(s1a_ref, s1b_ref, y1_ref, dinv_ref, b1_ref, w2_ref, y2_ref):
    dinv = dinv_ref[...]
    agg = s1a_ref[...] + s1b_ref[...] + y1_ref[...]
    h = jnp.maximum(agg * dinv + b1_ref[...], 0.0)
    y2_ref[...] = jnp.dot(h, w2_ref[...],
                          preferred_element_type=jnp.float32) * dinv


def _tc3a_body(s2a_ref, s2b_ref, y2_ref, dinv_ref, b2_ref, h2_ref, st_ref):
    i = pl.program_id(0)
    agg = s2a_ref[...] + s2b_ref[...] + y2_ref[...]
    h2 = jnp.maximum(agg * dinv_ref[...] + b2_ref[...], 0.0)
    h2_ref[...] = h2
    st = jnp.concatenate(
        [jnp.sum(h2, 0, keepdims=True), jnp.sum(h2 * h2, 0, keepdims=True)], 0)

    @pl.when(i == 0)
    def _():
        st_ref[...] = st

    @pl.when(i > 0)
    def _():
        st_ref[...] += st


def _tc3b_body(h2_ref, bat_ref, st_ref, g2_ref, beta2_ref, wfc_ref, bfc_ref,
               out_ref, pooled_s, cnt_s):
    i = pl.program_id(0)
    m = st_ref[0:1, :] * (1.0 / N)
    v = st_ref[1:2, :] * (1.0 / N) - m * m
    h2n = jnp.maximum(
        (h2_ref[...] - m) * lax.rsqrt(v + 1e-5) * g2_ref[...] + beta2_ref[...],
        0.0)
    oh = (bat_ref[...] == lax.broadcasted_iota(jnp.int32, (BLK, G), 1)
          ).astype(jnp.float32)
    p = lax.dot_general(oh, h2n, (((0,), (0,)), ((), ())),
                        preferred_element_type=jnp.float32)
    c = lax.dot_general(oh, jnp.ones((BLK, D), jnp.float32),
                        (((0,), (0,)), ((), ())),
                        preferred_element_type=jnp.float32)

    @pl.when(i == 0)
    def _():
        pooled_s[...] = p
        cnt_s[...] = c

    @pl.when(i > 0)
    def _():
        pooled_s[...] += p
        cnt_s[...] += c

    @pl.when(i == pl.num_programs(0) - 1)
    def _():
        pm = pooled_s[...] / jnp.maximum(cnt_s[...], 1.0)
        logits = jnp.dot(pm, wfc_ref[...],
                         preferred_element_type=jnp.float32) + bfc_ref[...]
        mx = jnp.max(logits, axis=1, keepdims=True)
        lse = jnp.log(jnp.sum(jnp.exp(logits - mx), 1, keepdims=True)) + mx
        out_ref[...] = logits - lse


def _row_spec(shape):
    return pl.BlockSpec(shape, lambda i: (i, 0))


def _const_spec(shape):
    return pl.BlockSpec(shape, lambda i: (0, 0))


def kernel(x, edge_index, batch, W1, b1, W2, b2, g2, beta2, Wfc, bfc):
    pad = EPAD - E
    srcp = jnp.concatenate(
        [edge_index[0], jnp.zeros((pad,), jnp.int32)]).reshape(NCHUNKS_P, CHUNK)
    dstp = jnp.concatenate(
        [edge_index[1], jnp.full((pad,), N, jnp.int32)]).reshape(NCHUNKS_P, CHUNK)

    srcf = srcp.reshape(EPAD)
    dstf = dstp.reshape(EPAD)
    cnt = _sc_degree(dstp)
    c0 = cnt[:N].reshape(N, 1)
    c1 = cnt[NPAD:NPAD + N].reshape(N, 1)

    y1, dinv = pl.pallas_call(
        _tc1_body,
        grid=(GRID,),
        in_specs=[_row_spec((BLK, 1)), _row_spec((BLK, 1)),
                  _row_spec((BLK, D)), _const_spec((D, D))],
        out_specs=[_row_spec((BLK, D)), _row_spec((BLK, 1))],
        out_shape=[jax.ShapeDtypeStruct((N, D), jnp.float32),
                   jax.ShapeDtypeStruct((N, 1), jnp.float32)],
    )(c0, c1, x, W1)

    s1 = _sc_msgpass(y1, srcf, dstf)

    y2 = pl.pallas_call(
        _tc2_body,
        grid=(GRID,),
        in_specs=[_row_spec((BLK, D)), _row_spec((BLK, D)),
                  _row_spec((BLK, D)), _row_spec((BLK, 1)),
                  _const_spec((1, D)), _const_spec((D, D))],
        out_specs=_row_spec((BLK, D)),
        out_shape=jax.ShapeDtypeStruct((N, D), jnp.float32),
    )(s1[:N], s1[NPAD:NPAD + N], y1, dinv, b1.reshape(1, D), W2)

    s2 = _sc_msgpass(y2, srcf, dstf)

    h2, st = pl.pallas_call(
        _tc3a_body,
        grid=(GRID,),
        in_specs=[_row_spec((BLK, D)), _row_spec((BLK, D)),
                  _row_spec((BLK, D)), _row_spec((BLK, 1)),
                  _const_spec((1, D))],
        out_specs=[_row_spec((BLK, D)), _const_spec((2, D))],
        out_shape=[jax.ShapeDtypeStruct((N, D), jnp.float32),
                   jax.ShapeDtypeStruct((2, D), jnp.float32)],
    )(s2[:N], s2[NPAD:NPAD + N], y2, dinv, b2.reshape(1, D))

    out = pl.pallas_call(
        _tc3b_body,
        grid=(GRID,),
        in_specs=[_row_spec((BLK, D)), _row_spec((BLK, 1)),
                  _const_spec((2, D)), _const_spec((1, D)),
                  _const_spec((1, D)), _const_spec((D, C)),
                  _const_spec((1, C))],
        out_specs=_const_spec((G, C)),
        out_shape=jax.ShapeDtypeStruct((G, C), jnp.float32),
        scratch_shapes=[pltpu.VMEM((G, D), jnp.float32),
                        pltpu.VMEM((G, D), jnp.float32)],
    )(h2, batch.reshape(N, 1), st, g2.reshape(1, D), beta2.reshape(1, D),
      Wfc, bfc.reshape(1, C))

    return out


# speedup vs baseline: 2.2069x; 1.8164x over previous
import functools

import jax
import jax.numpy as jnp
from jax import lax
from jax.experimental import pallas as pl
from jax.experimental.pallas import tpu as pltpu
from jax.experimental.pallas import tpu_sc as plsc

N = 10000
E = 320000
D = 128
C = 10
G = 64

NPAD = 10240
NC, NS = 2, 16
NW = NC * NS
CHUNK = 128
NCHUNKS = E // CHUNK
ROWS_PER_SUB = NPAD // NS
BLK = 1000
GRID = N // BLK

_mesh = plsc.VectorSubcoreMesh(core_axis_name="c", subcore_axis_name="s")


def _worker_id():
    return lax.axis_index("s") * NC + lax.axis_index("c")


def _num_chunks(w):
    return (NCHUNKS - w + NW - 1) // NW


@functools.partial(
    pl.kernel,
    out_type=jax.ShapeDtypeStruct((NC * NPAD,), jnp.float32),
    mesh=_mesh,
    scratch_types=[
        pltpu.VMEM((CHUNK,), jnp.int32),
        pltpu.VMEM((CHUNK,), jnp.float32),
        pltpu.VMEM((ROWS_PER_SUB,), jnp.float32),
        pltpu.VMEM_SHARED((NPAD,), jnp.float32),
    ],
)
def _sc_degree(dst_hbm, out_hbm, idx_v, ones_v, zbuf_v, cnt_sh):
    cid = lax.axis_index("c")
    sid = lax.axis_index("s")
    w = _worker_id()

    def fill(i, _):
        zbuf_v[pl.ds(i * 16, 16)] = jnp.zeros((16,), jnp.float32)
        ones_v[pl.ds((i % 8) * 16, 16)] = jnp.full((16,), 1.0, jnp.float32)
        return 0

    lax.fori_loop(0, ROWS_PER_SUB // 16, fill, 0)
    pltpu.sync_copy(zbuf_v, cnt_sh.at[pl.ds(sid * ROWS_PER_SUB, ROWS_PER_SUB)])
    plsc.subcore_barrier()

    def body(i, _):
        base = (w + i * NW) * CHUNK
        pltpu.sync_copy(dst_hbm.at[pl.ds(base, CHUNK)], idx_v)
        pltpu.sync_copy(ones_v, cnt_sh.at[idx_v], add=True)
        return 0

    lax.fori_loop(0, _num_chunks(w), body, 0)
    plsc.subcore_barrier()
    pltpu.sync_copy(
        cnt_sh.at[pl.ds(sid * ROWS_PER_SUB, ROWS_PER_SUB)],
        out_hbm.at[pl.ds(cid * NPAD + sid * ROWS_PER_SUB, ROWS_PER_SUB)],
    )


@functools.partial(
    pl.kernel,
    out_type=jax.ShapeDtypeStruct((NC * NPAD, D), jnp.float32),
    mesh=_mesh,
    scratch_types=[
        pltpu.VMEM((CHUNK,), jnp.int32),
        pltpu.VMEM((CHUNK,), jnp.int32),
        pltpu.VMEM((CHUNK, D), jnp.float32),
        pltpu.VMEM((CHUNK, D), jnp.float32),
        pltpu.VMEM_SHARED((NPAD, D), jnp.float32),
        pltpu.SemaphoreType.DMA,
    ],
)
def _sc_msgpass(y_hbm, src_hbm, dst_hbm, out_hbm,
                sidx_v, didx_v, rows_v, zbuf_v, acc_sh, sem):
    cid = lax.axis_index("c")
    sid = lax.axis_index("s")
    w = _worker_id()

    def fill(r, _):
        for c8 in range(D // 16):
            zbuf_v[r, pl.ds(c8 * 16, 16)] = jnp.zeros((16,), jnp.float32)
        return 0

    lax.fori_loop(0, CHUNK, fill, 0)
    for k in range(ROWS_PER_SUB // CHUNK):
        pltpu.sync_copy(
            zbuf_v, acc_sh.at[pl.ds(sid * ROWS_PER_SUB + k * CHUNK, CHUNK)])
    plsc.subcore_barrier()

    def body(i, _):
        base = (w + i * NW) * CHUNK
        pltpu.sync_copy(src_hbm.at[pl.ds(base, CHUNK)], sidx_v)
        pltpu.sync_copy(dst_hbm.at[pl.ds(base, CHUNK)], didx_v)
        pltpu.async_copy(y_hbm.at[sidx_v], rows_v, sem).wait()
        pltpu.sync_copy(rows_v, acc_sh.at[didx_v], add=True)
        return 0

    lax.fori_loop(0, _num_chunks(w), body, 0)
    plsc.subcore_barrier()
    for k in range(ROWS_PER_SUB // CHUNK):
        pltpu.sync_copy(
            acc_sh.at[pl.ds(sid * ROWS_PER_SUB + k * CHUNK, CHUNK)],
            out_hbm.at[pl.ds(cid * NPAD + sid * ROWS_PER_SUB + k * CHUNK,
                             CHUNK)],
        )


def _tc1_body(c0_ref, c1_ref, x_ref, w1_ref, y1_ref, dinv_ref):
    deg = c0_ref[...] + c1_ref[...] + 1.0
    dinv = lax.rsqrt(deg)
    xw = jnp.dot(x_ref[...], w1_ref[...], preferred_element_type=jnp.float32)
    y1_ref[...] = xw * dinv
    dinv_ref[...] = dinv


def _tc2_body(s1a_ref, s1b_ref, y1_ref, dinv_ref, b1_ref, w2_ref, y2_ref):
    dinv = dinv_ref[...]
    agg = s1a_ref[...] + s1b_ref[...] + y1_ref[...]
    h = jnp.maximum(agg * dinv + b1_ref[...], 0.0)
    y2_ref[...] = jnp.dot(h, w2_ref[...],
                          preferred_element_type=jnp.float32) * dinv


def _tc3a_body(s2a_ref, s2b_ref, y2_ref, dinv_ref, b2_ref, h2_ref, st_ref):
    i = pl.program_id(0)
    agg = s2a_ref[...] + s2b_ref[...] + y2_ref[...]
    h2 = jnp.maximum(agg * dinv_ref[...] + b2_ref[...], 0.0)
    h2_ref[...] = h2
    st = jnp.concatenate(
        [jnp.sum(h2, 0, keepdims=True), jnp.sum(h2 * h2, 0, keepdims=True)], 0)

    @pl.when(i == 0)
    def _():
        st_ref[...] = st

    @pl.when(i > 0)
    def _():
        st_ref[...] += st


def _tc3b_body(h2_ref, bat_ref, st_ref, g2_ref, beta2_ref, wfc_ref, bfc_ref,
               out_ref, pooled_s, cnt_s):
    i = pl.program_id(0)
    m = st_ref[0:1, :] * (1.0 / N)
    v = st_ref[1:2, :] * (1.0 / N) - m * m
    h2n = jnp.maximum(
        (h2_ref[...] - m) * lax.rsqrt(v + 1e-5) * g2_ref[...] + beta2_ref[...],
        0.0)
    oh = (bat_ref[...] == lax.broadcasted_iota(jnp.int32, (BLK, G), 1)
          ).astype(jnp.float32)
    p = lax.dot_general(oh, h2n, (((0,), (0,)), ((), ())),
                        preferred_element_type=jnp.float32)
    c = lax.dot_general(oh, jnp.ones((BLK, D), jnp.float32),
                        (((0,), (0,)), ((), ())),
                        preferred_element_type=jnp.float32)

    @pl.when(i == 0)
    def _():
        pooled_s[...] = p
        cnt_s[...] = c

    @pl.when(i > 0)
    def _():
        pooled_s[...] += p
        cnt_s[...] += c

    @pl.when(i == pl.num_programs(0) - 1)
    def _():
        pm = pooled_s[...] / jnp.maximum(cnt_s[...], 1.0)
        logits = jnp.dot(pm, wfc_ref[...],
                         preferred_element_type=jnp.float32) + bfc_ref[...]
        mx = jnp.max(logits, axis=1, keepdims=True)
        lse = jnp.log(jnp.sum(jnp.exp(logits - mx), 1, keepdims=True)) + mx
        out_ref[...] = logits - lse


def _row_spec(shape):
    return pl.BlockSpec(shape, lambda i: (i, 0))


def _const_spec(shape):
    return pl.BlockSpec(shape, lambda i: (0, 0))


def kernel(x, edge_index, batch, W1, b1, W2, b2, g2, beta2, Wfc, bfc):
    src = edge_index[0]
    dst = edge_index[1]

    cnt = _sc_degree(dst)
    c0 = cnt[:N].reshape(N, 1)
    c1 = cnt[NPAD:NPAD + N].reshape(N, 1)

    y1, dinv = pl.pallas_call(
        _tc1_body,
        grid=(GRID,),
        in_specs=[_row_spec((BLK, 1)), _row_spec((BLK, 1)),
                  _row_spec((BLK, D)), _const_spec((D, D))],
        out_specs=[_row_spec((BLK, D)), _row_spec((BLK, 1))],
        out_shape=[jax.ShapeDtypeStruct((N, D), jnp.float32),
                   jax.ShapeDtypeStruct((N, 1), jnp.float32)],
    )(c0, c1, x, W1)

    s1 = _sc_msgpass(y1, src, dst)

    y2 = pl.pallas_call(
        _tc2_body,
        grid=(GRID,),
        in_specs=[_row_spec((BLK, D)), _row_spec((BLK, D)),
                  _row_spec((BLK, D)), _row_spec((BLK, 1)),
                  _const_spec((1, D)), _const_spec((D, D))],
        out_specs=_row_spec((BLK, D)),
        out_shape=jax.ShapeDtypeStruct((N, D), jnp.float32),
    )(s1[:N], s1[NPAD:NPAD + N], y1, dinv, b1.reshape(1, D), W2)

    s2 = _sc_msgpass(y2, src, dst)

    h2, st = pl.pallas_call(
        _tc3a_body,
        grid=(GRID,),
        in_specs=[_row_spec((BLK, D)), _row_spec((BLK, D)),
                  _row_spec((BLK, D)), _row_spec((BLK, 1)),
                  _const_spec((1, D))],
        out_specs=[_row_spec((BLK, D)), _const_spec((2, D))],
        out_shape=[jax.ShapeDtypeStruct((N, D), jnp.float32),
                   jax.ShapeDtypeStruct((2, D), jnp.float32)],
    )(s2[:N], s2[NPAD:NPAD + N], y2, dinv, b2.reshape(1, D))

    out = pl.pallas_call(
        _tc3b_body,
        grid=(GRID,),
        in_specs=[_row_spec((BLK, D)), _row_spec((BLK, 1)),
                  _const_spec((2, D)), _const_spec((1, D)),
                  _const_spec((1, D)), _const_spec((D, C)),
                  _const_spec((1, C))],
        out_specs=_const_spec((G, C)),
        out_shape=jax.ShapeDtypeStruct((G, C), jnp.float32),
        scratch_shapes=[pltpu.VMEM((G, D), jnp.float32),
                        pltpu.VMEM((G, D), jnp.float32)],
    )(h2, batch.reshape(N, 1), st, g2.reshape(1, D), beta2.reshape(1, D),
      Wfc, bfc.reshape(1, C))

    return out
